# Initial kernel scaffold; baseline (speedup 1.0000x reference)
import jax
import jax.numpy as jnp
from jax.experimental import pallas as pl


def kernel(x, enc_W1, enc_b1, enc_W2, enc_b2, dec_W1, dec_b1, dec_W2, dec_b2, codebooks, gumbel, gumbel_t):
    raise NotImplementedError("write your pallas kernel here")


# speedup vs baseline: 2.4881x; 2.4881x over previous
import functools

import jax
import jax.numpy as jnp
from jax.experimental import pallas as pl

N_LAYERS = 3
COMMITMENT = 0.25


def _rqvae_kernel(x_ref, w1_ref, b1_ref, w2_ref, b2_ref,
                  dw1_ref, db1_ref, dw2_ref, db2_ref,
                  cb_ref, g_ref, out_ref):
    f32 = jnp.float32
    xb = x_ref[...]
    h = jnp.dot(xb, w1_ref[...], preferred_element_type=f32) + b1_ref[...]
    h = jnp.maximum(h, 0.0)
    res = jnp.dot(h, w2_ref[...], preferred_element_type=f32) + b2_ref[...]
    res0 = res

    T = xb.shape[0]
    K = cb_ref.shape[1]
    iota = jax.lax.broadcasted_iota(jnp.int32, (T, K), 1)

    rq = f32(0.0)
    for i in range(N_LAYERS):
        cb = cb_ref[i]
        cbsq = jnp.sum(cb * cb, axis=-1)
        q = jnp.sum(res * res, axis=-1, keepdims=True)
        prod = jax.lax.dot_general(
            res, cb, (((1,), (1,)), ((), ())), preferred_element_type=f32)
        dist = q - 2.0 * prod + cbsq[None, :]
        s = g_ref[i] - dist
        m = jnp.max(s, axis=-1, keepdims=True)
        idx = jnp.min(jnp.where(s == m, iota, K), axis=-1, keepdims=True)
        onehot = (iota == idx).astype(f32)
        emb = jnp.dot(onehot, cb, preferred_element_type=f32)
        res = res - emb
        rq = rq + jnp.sum(res * res)

    esum = res0 - res
    h2 = jnp.dot(esum, dw1_ref[...], preferred_element_type=f32) + db1_ref[...]
    h2 = jnp.maximum(h2, 0.0)
    x_hat = jnp.dot(h2, dw2_ref[...], preferred_element_type=f32) + db2_ref[...]
    recon = jnp.sum((x_hat - xb) ** 2)

    total = recon + (1.0 + COMMITMENT) * rq

    @pl.when(pl.program_id(0) == 0)
    def _():
        out_ref[...] = jnp.zeros_like(out_ref)

    out_ref[...] = out_ref[...] + total


@functools.partial(jax.jit, static_argnames=("tile",))
def _run(x, enc_W1, enc_b1, enc_W2, enc_b2, dec_W1, dec_b1, dec_W2, dec_b2,
         codebooks, gumbel, tile):
    B, IN = x.shape
    HID = enc_W1.shape[1]
    L, K, D = codebooks.shape
    grid = (B // tile,)
    const = lambda shape: pl.BlockSpec(shape, lambda i: (0,) * len(shape))
    out = pl.pallas_call(
        _rqvae_kernel,
        grid=grid,
        in_specs=[
            pl.BlockSpec((tile, IN), lambda i: (i, 0)),
            const((IN, HID)), const((HID,)),
            const((HID, D)), const((D,)),
            const((D, HID)), const((HID,)),
            const((HID, IN)), const((IN,)),
            const((L, K, D)),
            pl.BlockSpec((L, tile, K), lambda i: (0, i, 0)),
        ],
        out_specs=pl.BlockSpec((1, 1), lambda i: (0, 0)),
        out_shape=jax.ShapeDtypeStruct((1, 1), jnp.float32),
    )(x, enc_W1, enc_b1, enc_W2, enc_b2, dec_W1, dec_b1, dec_W2, dec_b2,
      codebooks, gumbel)
    return out[0, 0] / B


def kernel(x, enc_W1, enc_b1, enc_W2, enc_b2, dec_W1, dec_b1, dec_W2, dec_b2,
           codebooks, gumbel, gumbel_t):
    del gumbel_t
    tile = 256 if x.shape[0] % 256 == 0 else x.shape[0]
    return _run(x, enc_W1, enc_b1, enc_W2, enc_b2,
                dec_W1, dec_b1, dec_W2, dec_b2, codebooks, gumbel, tile)
